# baseline (device time: 203871 ns/iter reference)
import jax
import jax.numpy as jnp
from jax import lax
from jax.experimental import pallas as pl
from jax.experimental.pallas import tpu as pltpu

N_DEV = 4
SQ = 2048
SKV = 2048
HQ = 32
HQ_LOC = HQ // N_DEV
DH = 128
D_MODEL = 1024
BLK = 64
QB = 256
R = SQ // N_DEV
NP = 4
PC = D_MODEL // NP
SCALE = 0.08838834764831843
BF = jnp.bfloat16
F32 = jnp.float32


def _fused(xb, wq_loc, k, v, wo_loc):


    def body(x_ref, wq_ref, k_ref, v_ref, wo_ref, out_ref,
             q_scr, ctx_scr, rs_send, rs_recv, ag_buf,
             rs_send_sems, rs_recv_sems, ag_send_sems, ag_recv_sems):
        my = lax.axis_index("i")
        left = (my - 1) % N_DEV
        right = (my + 1) % N_DEV

        barrier_sem = pltpu.get_barrier_semaphore()
        for nbr in (left, right):
            pl.semaphore_signal(
                barrier_sem, inc=1,
                device_id=(nbr,), device_id_type=pl.DeviceIdType.MESH,
            )
        pl.semaphore_wait(barrier_sem, 2)

        def rs_rdma(t, p):
            return pltpu.make_async_remote_copy(
                src_ref=rs_send.at[t, p],
                dst_ref=rs_recv.at[t, p],
                send_sem=rs_send_sems.at[t, p],
                recv_sem=rs_recv_sems.at[t, p],
                device_id=(right,),
                device_id_type=pl.DeviceIdType.MESH,
            )

        def ag_rdma(u, p):
            return pltpu.make_async_remote_copy(
                src_ref=ag_buf.at[u, p],
                dst_ref=ag_buf.at[u + 1, p],
                send_sem=ag_send_sems.at[u, p],
                recv_sem=ag_recv_sems.at[u, p],
                device_id=(right,),
                device_id_type=pl.DeviceIdType.MESH,
            )

        def pcols(p):
            return pl.ds(p * PC, PC)

        band_mask = (
            lax.broadcasted_iota(jnp.int32, (QB, QB), 0) // BLK
            >= lax.broadcasted_iota(jnp.int32, (QB, QB), 1) // BLK
        )

        for j in range(N_DEV):
            c = (my - j) % N_DEV
            for sub in range(2):
                qi = c * 2 + sub
                rows = pl.ds(qi * QB, QB)
                q_scr[...] = (jnp.dot(
                    x_ref[rows, :], wq_ref[...], preferred_element_type=F32
                ) * SCALE).astype(BF)
                for h in range(HQ_LOC):
                    hc = pl.ds(h * DH, DH)
                    q_h = q_scr[:, hc]

                    def kv_body(t, carry, q_h=q_h, hc=hc, qi=qi):
                        ctx, den = carry
                        kv = pl.ds(t * QB, QB)
                        s = lax.dot_general(
                            q_h, k_ref[kv, hc],
                            (((1,), (1,)), ((), ())),
                            preferred_element_type=F32,
                        )
                        e = jnp.where(
                            jnp.logical_or(t < qi, band_mask),
                            jnp.exp(s.astype(BF)), jnp.array(0, BF),
                        )
                        den = den + jnp.sum(e, axis=1, keepdims=True,
                                            dtype=F32)
                        ctx = ctx + lax.dot_general(
                            e, v_ref[kv, hc],
                            (((1,), (0,)), ((), ())),
                            preferred_element_type=F32,
                        )
                        return ctx, den

                    ctx, den = lax.fori_loop(
                        0, qi + 1, kv_body,
                        (jnp.zeros((QB, DH), F32), jnp.zeros((QB, 1), F32)),
                    )
                    ctx_scr[:, hc] = (ctx / den).astype(BF)
                out_ref[rows, :] = jnp.dot(
                    ctx_scr[...], wo_ref[...], preferred_element_type=F32
                )

            rows_c = pl.ds(c * R, R)
            if j == 0:
                for p in range(NP):
                    rs_send[0, p] = out_ref[rows_c, pcols(p)].astype(BF)
                    rs_rdma(0, p).start()
            elif j < N_DEV - 1:
                for p in range(NP):
                    rs_rdma(j - 1, p).wait_recv()
                    rs_send[j, p] = (
                        out_ref[rows_c, pcols(p)]
                        + rs_recv[j - 1, p][...].astype(F32)
                    ).astype(BF)
                    rs_rdma(j, p).start()
            else:
                for p in range(NP):
                    rs_rdma(N_DEV - 2, p).wait_recv()
                    red = (
                        out_ref[rows_c, pcols(p)]
                        + rs_recv[N_DEV - 2, p][...].astype(F32)
                    )
                    out_ref[rows_c, pcols(p)] = red
                    ag_buf[0, p] = red.astype(BF)
                    ag_rdma(0, p).start()

        for u in range(1, N_DEV - 1):
            rows_u = pl.ds(((my - u + 1) % N_DEV) * R, R)
            for p in range(NP):
                ag_rdma(u - 1, p).wait_recv()
                ag_rdma(u, p).start()
                out_ref[rows_u, pcols(p)] = ag_buf[u, p][...].astype(F32)
        rows_last = pl.ds(((my - (N_DEV - 2)) % N_DEV) * R, R)
        for p in range(NP):
            ag_rdma(N_DEV - 2, p).wait_recv()
            out_ref[rows_last, pcols(p)] = ag_buf[N_DEV - 1, p][...].astype(F32)

        for t in range(N_DEV - 1):
            for p in range(NP):
                rs_rdma(t, p).wait_send()
                ag_rdma(t, p).wait_send()

    return pl.pallas_call(
        body,
        out_shape=jax.ShapeDtypeStruct((SQ, D_MODEL), F32),
        in_specs=[pl.BlockSpec(memory_space=pltpu.VMEM)] * 5,
        out_specs=pl.BlockSpec(memory_space=pltpu.VMEM),
        scratch_shapes=[
            pltpu.VMEM((QB, HQ_LOC * DH), BF),
            pltpu.VMEM((QB, HQ_LOC * DH), BF),
            pltpu.VMEM((N_DEV - 1, NP, R, PC), BF),
            pltpu.VMEM((N_DEV - 1, NP, R, PC), BF),
            pltpu.VMEM((N_DEV, NP, R, PC), BF),
            pltpu.SemaphoreType.DMA((N_DEV - 1, NP)),
            pltpu.SemaphoreType.DMA((N_DEV - 1, NP)),
            pltpu.SemaphoreType.DMA((N_DEV - 1, NP)),
            pltpu.SemaphoreType.DMA((N_DEV - 1, NP)),
        ],
        compiler_params=pltpu.CompilerParams(
            collective_id=0, vmem_limit_bytes=48 * 1024 * 1024
        ),
    )(xb, wq_loc, k, v, wo_loc)


def kernel(x, Wq, K_ext, V_ext, Wo):
    my = lax.axis_index("i")

    xb = x[0].astype(BF)
    Wq_loc = lax.dynamic_slice_in_dim(
        Wq.reshape(D_MODEL, HQ, DH), my * HQ_LOC, HQ_LOC, axis=1
    ).reshape(D_MODEL, HQ_LOC * DH).astype(BF)
    k = K_ext[0].reshape(SKV, HQ_LOC * DH).astype(BF)
    v = V_ext[0].reshape(SKV, HQ_LOC * DH).astype(BF)
    Wo_loc = lax.dynamic_slice_in_dim(
        Wo, my * HQ_LOC * DH, HQ_LOC * DH, axis=0
    ).astype(BF)

    out = _fused(xb, Wq_loc, k, v, Wo_loc)
    return out[None]


# device time: 179484 ns/iter; 1.1359x vs baseline; 1.1359x over previous
import jax
import jax.numpy as jnp
from jax import lax
from jax.experimental import pallas as pl
from jax.experimental.pallas import tpu as pltpu

N_DEV = 4
SQ = 2048
SKV = 2048
HQ = 32
HQ_LOC = HQ // N_DEV
DH = 128
D_MODEL = 1024
BLK = 64
QB = 512
R = SQ // N_DEV
NP = 4
PC = D_MODEL // NP
SCALE = 0.08838834764831843
BF = jnp.bfloat16
F32 = jnp.float32


def _fused(xb, wq_loc, k, v, wo_loc):


    def body(x_ref, wq_ref, k_ref, v_ref, wo_ref, out_ref,
             q_scr, ctx_scr, rs_send, rs_recv, ag_buf,
             rs_send_sems, rs_recv_sems, ag_send_sems, ag_recv_sems):
        my = lax.axis_index("i")
        left = (my - 1) % N_DEV
        right = (my + 1) % N_DEV

        barrier_sem = pltpu.get_barrier_semaphore()
        for nbr in (left, right):
            pl.semaphore_signal(
                barrier_sem, inc=1,
                device_id=(nbr,), device_id_type=pl.DeviceIdType.MESH,
            )
        pl.semaphore_wait(barrier_sem, 2)

        def rs_rdma(t, p):
            return pltpu.make_async_remote_copy(
                src_ref=rs_send.at[t, p],
                dst_ref=rs_recv.at[t, p],
                send_sem=rs_send_sems.at[t, p],
                recv_sem=rs_recv_sems.at[t, p],
                device_id=(right,),
                device_id_type=pl.DeviceIdType.MESH,
            )

        def ag_rdma(u, p):
            return pltpu.make_async_remote_copy(
                src_ref=ag_buf.at[u, p],
                dst_ref=ag_buf.at[u + 1, p],
                send_sem=ag_send_sems.at[u, p],
                recv_sem=ag_recv_sems.at[u, p],
                device_id=(right,),
                device_id_type=pl.DeviceIdType.MESH,
            )

        def pcols(p):
            return pl.ds(p * PC, PC)

        band_mask = (
            lax.broadcasted_iota(jnp.int32, (QB, QB), 0) // BLK
            >= lax.broadcasted_iota(jnp.int32, (QB, QB), 1) // BLK
        )

        def compute_chunk(cc):
            if True:
                qi = cc
                rows = pl.ds(qi * QB, QB)
                L0 = qi * QB
                q_scr[...] = (jnp.dot(
                    x_ref[rows, :], wq_ref[...], preferred_element_type=F32
                ) * SCALE).astype(BF)
                for h in range(HQ_LOC):
                    hc = pl.ds(h * DH, DH)
                    q_h = q_scr[:, hc]
                    s_diag = lax.dot_general(
                        q_h, k_ref[pl.ds(L0, QB), hc],
                        (((1,), (1,)), ((), ())),
                        preferred_element_type=F32,
                    )
                    e_diag = jnp.where(
                        band_mask, jnp.exp(s_diag.astype(BF)),
                        jnp.array(0, BF),
                    )
                    den = jnp.sum(e_diag, axis=1, keepdims=True, dtype=F32)
                    ctx = lax.dot_general(
                        e_diag, v_ref[pl.ds(L0, QB), hc],
                        (((1,), (0,)), ((), ())),
                        preferred_element_type=F32,
                    )
                    if L0 > 0:
                        s_full = lax.dot_general(
                            q_h, k_ref[pl.ds(0, L0), hc],
                            (((1,), (1,)), ((), ())),
                            preferred_element_type=F32,
                        )
                        e_full = jnp.exp(s_full.astype(BF))
                        den += jnp.sum(e_full, axis=1, keepdims=True,
                                       dtype=F32)
                        ctx += lax.dot_general(
                            e_full, v_ref[pl.ds(0, L0), hc],
                            (((1,), (0,)), ((), ())),
                            preferred_element_type=F32,
                        )
                    ctx_scr[:, hc] = (ctx / den).astype(BF)
                out_ref[rows, :] = jnp.dot(
                    ctx_scr[...], wo_ref[...], preferred_element_type=F32
                )

        for j in range(N_DEV):
            c = (my - j) % N_DEV
            for cc in range(N_DEV):
                @pl.when(c == cc)
                def _(cc=cc):
                    compute_chunk(cc)

            rows_c = pl.ds(c * R, R)
            if j == 0:
                for p in range(NP):
                    rs_send[0, p] = out_ref[rows_c, pcols(p)].astype(BF)
                    rs_rdma(0, p).start()
            elif j < N_DEV - 1:
                for p in range(NP):
                    rs_rdma(j - 1, p).wait_recv()
                    rs_send[j, p] = (
                        out_ref[rows_c, pcols(p)]
                        + rs_recv[j - 1, p][...].astype(F32)
                    ).astype(BF)
                    rs_rdma(j, p).start()
            else:
                for p in range(NP):
                    rs_rdma(N_DEV - 2, p).wait_recv()
                    red = (
                        out_ref[rows_c, pcols(p)]
                        + rs_recv[N_DEV - 2, p][...].astype(F32)
                    )
                    out_ref[rows_c, pcols(p)] = red
                    ag_buf[0, p] = red.astype(BF)
                    ag_rdma(0, p).start()

        for u in range(1, N_DEV - 1):
            rows_u = pl.ds(((my - u + 1) % N_DEV) * R, R)
            for p in range(NP):
                ag_rdma(u - 1, p).wait_recv()
                ag_rdma(u, p).start()
                out_ref[rows_u, pcols(p)] = ag_buf[u, p][...].astype(F32)
        rows_last = pl.ds(((my - (N_DEV - 2)) % N_DEV) * R, R)
        for p in range(NP):
            ag_rdma(N_DEV - 2, p).wait_recv()
            out_ref[rows_last, pcols(p)] = ag_buf[N_DEV - 1, p][...].astype(F32)

        for t in range(N_DEV - 1):
            for p in range(NP):
                rs_rdma(t, p).wait_send()
                ag_rdma(t, p).wait_send()

    return pl.pallas_call(
        body,
        out_shape=jax.ShapeDtypeStruct((SQ, D_MODEL), F32),
        in_specs=[pl.BlockSpec(memory_space=pltpu.VMEM)] * 5,
        out_specs=pl.BlockSpec(memory_space=pltpu.VMEM),
        scratch_shapes=[
            pltpu.VMEM((QB, HQ_LOC * DH), BF),
            pltpu.VMEM((QB, HQ_LOC * DH), BF),
            pltpu.VMEM((N_DEV - 1, NP, R, PC), BF),
            pltpu.VMEM((N_DEV - 1, NP, R, PC), BF),
            pltpu.VMEM((N_DEV, NP, R, PC), BF),
            pltpu.SemaphoreType.DMA((N_DEV - 1, NP)),
            pltpu.SemaphoreType.DMA((N_DEV - 1, NP)),
            pltpu.SemaphoreType.DMA((N_DEV - 1, NP)),
            pltpu.SemaphoreType.DMA((N_DEV - 1, NP)),
        ],
        compiler_params=pltpu.CompilerParams(
            collective_id=0, vmem_limit_bytes=48 * 1024 * 1024
        ),
    )(xb, wq_loc, k, v, wo_loc)


def kernel(x, Wq, K_ext, V_ext, Wo):
    my = lax.axis_index("i")

    xb = x[0].astype(BF)
    Wq_loc = lax.dynamic_slice_in_dim(
        Wq.reshape(D_MODEL, HQ, DH), my * HQ_LOC, HQ_LOC, axis=1
    ).reshape(D_MODEL, HQ_LOC * DH).astype(BF)
    k = K_ext[0].reshape(SKV, HQ_LOC * DH).astype(BF)
    v = V_ext[0].reshape(SKV, HQ_LOC * DH).astype(BF)
    Wo_loc = lax.dynamic_slice_in_dim(
        Wo, my * HQ_LOC * DH, HQ_LOC * DH, axis=0
    ).astype(BF)

    out = _fused(xb, Wq_loc, k, v, Wo_loc)
    return out[None]


# device time: 144994 ns/iter; 1.4061x vs baseline; 1.2379x over previous
import jax
import jax.numpy as jnp
from jax import lax
from jax.experimental import pallas as pl
from jax.experimental.pallas import tpu as pltpu

N_DEV = 4
SQ = 2048
SKV = 2048
HQ = 32
HQ_LOC = HQ // N_DEV
DH = 128
D_MODEL = 1024
BLK = 64
QB = 256
R = SQ // N_DEV
NP = 4
PC = D_MODEL // NP
SCALE = 0.08838834764831843
BF = jnp.bfloat16
F32 = jnp.float32


def _fused(xb, wq_loc, k, v, wo_loc):

    def body(x_ref, wq_ref, k_ref, v_ref, wo_ref, out_ref,
             q_scr, ctx_scr, rs_send, rs_recv, ag_buf,
             rs_send_sems, rs_recv_sems, ag_send_sems, ag_recv_sems):
        my = lax.axis_index("i")
        left = (my - 1) % N_DEV
        right = (my + 1) % N_DEV

        barrier_sem = pltpu.get_barrier_semaphore()
        for nbr in (left, right):
            pl.semaphore_signal(
                barrier_sem, inc=1,
                device_id=(nbr,), device_id_type=pl.DeviceIdType.MESH,
            )
        pl.semaphore_wait(barrier_sem, 2)

        for qi in range(SQ // QB):
            rows = pl.ds(qi * QB, QB)
            L0 = qi * QB
            L = L0 + QB

            q_scr[...] = (jnp.dot(
                x_ref[rows, :], wq_ref[...], preferred_element_type=F32
            ) * SCALE).astype(BF)

            band_mask = (
                lax.broadcasted_iota(jnp.int32, (QB, QB), 0) // BLK
                >= lax.broadcasted_iota(jnp.int32, (QB, QB), 1) // BLK
            )

            for h in range(HQ_LOC):
                q_h = q_scr[:, h * DH:(h + 1) * DH]
                s_diag = lax.dot_general(
                    q_h, k_ref[h, L0:L, :],
                    (((1,), (1,)), ((), ())),
                    preferred_element_type=F32,
                )
                e_diag = jnp.where(band_mask, jnp.exp(s_diag), 0.0)
                den = jnp.sum(e_diag, axis=1, keepdims=True)
                ctx = lax.dot_general(
                    e_diag.astype(BF), v_ref[h, L0:L, :],
                    (((1,), (0,)), ((), ())),
                    preferred_element_type=F32,
                )
                if L0 > 0:
                    s_full = lax.dot_general(
                        q_h, k_ref[h, :L0, :],
                        (((1,), (1,)), ((), ())),
                        preferred_element_type=F32,
                    )
                    e_full = jnp.exp(s_full)
                    den += jnp.sum(e_full, axis=1, keepdims=True)
                    ctx += lax.dot_general(
                        e_full.astype(BF), v_ref[h, :L0, :],
                        (((1,), (0,)), ((), ())),
                        preferred_element_type=F32,
                    )
                ctx_scr[:, h * DH:(h + 1) * DH] = (ctx / den).astype(BF)

            out_ref[rows, :] = jnp.dot(
                ctx_scr[...], wo_ref[...], preferred_element_type=F32
            )

        def rs_rdma(t, p):
            return pltpu.make_async_remote_copy(
                src_ref=rs_send.at[t, p],
                dst_ref=rs_recv.at[t, p],
                send_sem=rs_send_sems.at[t, p],
                recv_sem=rs_recv_sems.at[t, p],
                device_id=(right,),
                device_id_type=pl.DeviceIdType.MESH,
            )

        def ag_rdma(u, p):
            return pltpu.make_async_remote_copy(
                src_ref=ag_buf.at[u, p],
                dst_ref=ag_buf.at[u + 1, p],
                send_sem=ag_send_sems.at[u, p],
                recv_sem=ag_recv_sems.at[u, p],
                device_id=(right,),
                device_id_type=pl.DeviceIdType.MESH,
            )

        def pcols(p):
            return pl.ds(p * PC, PC)

        rows0 = pl.ds(((my - 0) % N_DEV) * R, R)
        for p in range(NP):
            rs_send[0, p] = out_ref[rows0, pcols(p)].astype(BF)
            rs_rdma(0, p).start()
        for t in range(1, N_DEV - 1):
            rows_t = pl.ds(((my - t) % N_DEV) * R, R)
            for p in range(NP):
                rs_rdma(t - 1, p).wait_recv()
                rs_send[t, p] = (
                    out_ref[rows_t, pcols(p)]
                    + rs_recv[t - 1, p][...].astype(F32)
                ).astype(BF)
                rs_rdma(t, p).start()

        rows_m = pl.ds(((my + 1) % N_DEV) * R, R)
        for p in range(NP):
            rs_rdma(N_DEV - 2, p).wait_recv()
            red = (
                out_ref[rows_m, pcols(p)]
                + rs_recv[N_DEV - 2, p][...].astype(F32)
            )
            out_ref[rows_m, pcols(p)] = red
            ag_buf[0, p] = red.astype(BF)
            ag_rdma(0, p).start()

        for u in range(1, N_DEV - 1):
            rows_u = pl.ds(((my - u + 1) % N_DEV) * R, R)
            for p in range(NP):
                ag_rdma(u - 1, p).wait_recv()
                ag_rdma(u, p).start()
                out_ref[rows_u, pcols(p)] = ag_buf[u, p][...].astype(F32)
        rows_last = pl.ds(((my - (N_DEV - 2)) % N_DEV) * R, R)
        for p in range(NP):
            ag_rdma(N_DEV - 2, p).wait_recv()
            out_ref[rows_last, pcols(p)] = ag_buf[N_DEV - 1, p][...].astype(F32)

        for t in range(N_DEV - 1):
            for p in range(NP):
                rs_rdma(t, p).wait_send()
                ag_rdma(t, p).wait_send()

    return pl.pallas_call(
        body,
        out_shape=jax.ShapeDtypeStruct((SQ, D_MODEL), F32),
        in_specs=[pl.BlockSpec(memory_space=pltpu.VMEM)] * 5,
        out_specs=pl.BlockSpec(memory_space=pltpu.VMEM),
        scratch_shapes=[
            pltpu.VMEM((QB, HQ_LOC * DH), BF),
            pltpu.VMEM((QB, HQ_LOC * DH), BF),
            pltpu.VMEM((N_DEV - 1, NP, R, PC), BF),
            pltpu.VMEM((N_DEV - 1, NP, R, PC), BF),
            pltpu.VMEM((N_DEV, NP, R, PC), BF),
            pltpu.SemaphoreType.DMA((N_DEV - 1, NP)),
            pltpu.SemaphoreType.DMA((N_DEV - 1, NP)),
            pltpu.SemaphoreType.DMA((N_DEV - 1, NP)),
            pltpu.SemaphoreType.DMA((N_DEV - 1, NP)),
        ],
        compiler_params=pltpu.CompilerParams(
            collective_id=0, vmem_limit_bytes=48 * 1024 * 1024
        ),
    )(xb, wq_loc, k, v, wo_loc)


def kernel(x, Wq, K_ext, V_ext, Wo):
    my = lax.axis_index("i")

    xb = x[0].astype(BF)
    Wq_loc = lax.dynamic_slice_in_dim(
        Wq.reshape(D_MODEL, HQ, DH), my * HQ_LOC, HQ_LOC, axis=1
    ).reshape(D_MODEL, HQ_LOC * DH).astype(BF)
    k = K_ext[0].transpose(1, 0, 2).astype(BF)
    v = V_ext[0].transpose(1, 0, 2).astype(BF)
    Wo_loc = lax.dynamic_slice_in_dim(
        Wo, my * HQ_LOC * DH, HQ_LOC * DH, axis=0
    ).astype(BF)

    out = _fused(xb, Wq_loc, k, v, Wo_loc)
    return out[None]
